# baseline (speedup 1.0000x reference)
import functools

import jax
import jax.numpy as jnp
from jax import lax
from jax.experimental import pallas as pl
from jax.experimental.pallas import tpu as pltpu
from jax.experimental.pallas import tpu_sc as plsc

D_MODEL = 2048
NUM_EXPERTS = 64
TOP_K = 8
N_TOKENS = 16384

BLK = 1024
N_CHUNKS = 1

_NC = 2
_NS = 16
_NW = _NC * _NS
_L = 16


def _router_body(x_ref, w_ref, logits_ref, scores_ref):
    logits = jnp.dot(x_ref[...], w_ref[...], preferred_element_type=jnp.float32)
    logits_ref[...] = logits
    m = jnp.max(logits, axis=-1, keepdims=True)
    e = jnp.exp(logits - m)
    scores_ref[...] = e / jnp.sum(e, axis=-1, keepdims=True)


def _tc_router(x, W_router):
    n_rows = x.shape[0]
    n_blocks = n_rows // BLK
    return pl.pallas_call(
        _router_body,
        grid=(n_blocks,),
        in_specs=[
            pl.BlockSpec((BLK, D_MODEL), lambda i: (i, 0)),
            pl.BlockSpec((D_MODEL, NUM_EXPERTS), lambda i: (0, 0)),
        ],
        out_specs=[
            pl.BlockSpec((BLK, NUM_EXPERTS), lambda i: (i, 0)),
            pl.BlockSpec((BLK, NUM_EXPERTS), lambda i: (i, 0)),
        ],
        out_shape=[
            jax.ShapeDtypeStruct((n_rows, NUM_EXPERTS), jnp.float32),
            jax.ShapeDtypeStruct((n_rows, NUM_EXPERTS), jnp.float32),
        ],
    )(x, W_router)


def _make_sc_topk_body(rows_per_w):
    def _sc_topk_body(scores_hbm, ew_hbm, ei_hbm, cnt_hbm, s_v, ew_v, ei_v,
                      hist_v, sem):
        c = lax.axis_index("c")
        s_id = lax.axis_index("s")
        wid = s_id * _NC + c
        base = wid * rows_per_w

        pltpu.sync_copy(scores_hbm.at[pl.ds(base, rows_per_w)], s_v)

        iota = lax.iota(jnp.int32, _L)
        lane_lt8 = iota < TOP_K
        zeros16 = jnp.zeros((_L,), jnp.int32)
        ones16 = jnp.ones((_L,), jnp.int32)
        for j in range(NUM_EXPERTS // _L):
            hist_v[pl.ds(_L * j, _L)] = zeros16

        def merge(ka, va, kb, vb, descending):
            mk = jnp.where(lane_lt8, ka, kb)
            mv = jnp.where(lane_lt8, va, vb)
            return plsc.sort_key_val(mk, mv, descending=descending)

        @plsc.parallel_loop(0, rows_per_w, 1, unroll=8)
        def row_body(r):
            ks, vs = [], []
            for j in range(NUM_EXPERTS // _L):
                kj = s_v[r, pl.ds(_L * j, _L)]
                sk, sv = plsc.sort_key_val(
                    kj, iota + _L * j, descending=(j % 2 == 0)
                )
                ks.append(sk)
                vs.append(sv)
            k01, v01 = merge(ks[0], vs[0], ks[1], vs[1], True)
            k23, v23 = merge(ks[2], vs[2], ks[3], vs[3], False)
            kf, vf = merge(k01, v01, k23, v23, True)
            out_idx = r * TOP_K + iota
            plsc.store_scatter(ew_v, [out_idx], kf, mask=lane_lt8)
            plsc.store_scatter(ei_v, [out_idx], vf, mask=lane_lt8)

        def hist_body(i, carry):
            r16 = i * (4 * _L)
            for u in range(4):
                v = ei_v[pl.ds(r16 + u * _L, _L)]
                plsc.addupdate_scatter(hist_v, [v], ones16)
            return carry

        lax.fori_loop(0, rows_per_w * TOP_K // (4 * _L), hist_body, 0)

        pltpu.sync_copy(ew_v, ew_hbm.at[pl.ds(base * TOP_K, rows_per_w * TOP_K)])
        pltpu.sync_copy(ei_v, ei_hbm.at[pl.ds(base * TOP_K, rows_per_w * TOP_K)])
        pltpu.sync_copy(hist_v, cnt_hbm.at[wid])

    return _sc_topk_body


@functools.cache
def _sc_topk(n_rows):
    rows_per_w = n_rows // _NW
    return pl.kernel(
        _make_sc_topk_body(rows_per_w),
        out_type=[
            jax.ShapeDtypeStruct((n_rows * TOP_K,), jnp.float32),
            jax.ShapeDtypeStruct((n_rows * TOP_K,), jnp.int32),
            jax.ShapeDtypeStruct((_NW, NUM_EXPERTS), jnp.int32),
        ],
        mesh=plsc.VectorSubcoreMesh(
            core_axis_name="c", subcore_axis_name="s",
            num_cores=_NC, num_subcores=_NS,
        ),
        compiler_params=pltpu.CompilerParams(needs_layout_passes=False),
        scratch_types=[
            pltpu.VMEM((rows_per_w, NUM_EXPERTS), jnp.float32),
            pltpu.VMEM((rows_per_w * TOP_K,), jnp.float32),
            pltpu.VMEM((rows_per_w * TOP_K,), jnp.int32),
            pltpu.VMEM((NUM_EXPERTS,), jnp.int32),
            pltpu.SemaphoreType.DMA,
        ],
    )


def kernel(x, W_router):
    chunk = N_TOKENS // N_CHUNKS
    louts, souts, ews, eis, cnts = [], [], [], [], []
    for i in range(N_CHUNKS):
        lg, sc = _tc_router(lax.slice(x, (i * chunk, 0), ((i + 1) * chunk, D_MODEL)), W_router)
        ew_flat, ei_flat, cnt_p = _sc_topk(chunk)(sc)
        louts.append(lg)
        souts.append(sc)
        ews.append(ew_flat.reshape(chunk, TOP_K))
        eis.append(ei_flat.reshape(chunk, TOP_K))
        cnts.append(cnt_p)
    if N_CHUNKS == 1:
        logits, scores, ew, ei = louts[0], souts[0], ews[0], eis[0]
    else:
        logits = jnp.concatenate(louts, axis=0)
        scores = jnp.concatenate(souts, axis=0)
        ew = jnp.concatenate(ews, axis=0)
        ei = jnp.concatenate(eis, axis=0)
    cnt = jnp.sum(jnp.stack(cnts), axis=(0, 1), dtype=jnp.int32)
    one_f = W_router[0, 0] * 0.0 + 1.0
    one_i = one_f.astype(jnp.int32)
    return (logits * one_f, scores * one_f, ew * one_f, ei * one_i, cnt * one_i)


# speedup vs baseline: 1.1258x; 1.1258x over previous
import functools

import jax
import jax.numpy as jnp
from jax import lax
from jax.experimental import pallas as pl
from jax.experimental.pallas import tpu as pltpu
from jax.experimental.pallas import tpu_sc as plsc

D_MODEL = 2048
NUM_EXPERTS = 64
TOP_K = 8
N_TOKENS = 16384

BLK = 1024
N_CHUNKS = 1

_NC = 2
_NS = 16
_NW = _NC * _NS
_L = 16


def _router_body(x_ref, w_ref, logits_ref, scores_ref):
    logits = jnp.dot(x_ref[...], w_ref[...], preferred_element_type=jnp.float32)
    logits_ref[...] = logits
    m = jnp.max(logits, axis=-1, keepdims=True)
    e = jnp.exp(logits - m)
    scores_ref[...] = e / jnp.sum(e, axis=-1, keepdims=True)


def _tc_router(x, W_router):
    n_rows = x.shape[0]
    n_blocks = n_rows // BLK
    return pl.pallas_call(
        _router_body,
        grid=(n_blocks,),
        in_specs=[
            pl.BlockSpec((BLK, D_MODEL), lambda i: (i, 0)),
            pl.BlockSpec((D_MODEL, NUM_EXPERTS), lambda i: (0, 0)),
        ],
        out_specs=[
            pl.BlockSpec((BLK, NUM_EXPERTS), lambda i: (i, 0)),
            pl.BlockSpec((BLK, NUM_EXPERTS), lambda i: (i, 0)),
        ],
        out_shape=[
            jax.ShapeDtypeStruct((n_rows, NUM_EXPERTS), jnp.float32),
            jax.ShapeDtypeStruct((n_rows, NUM_EXPERTS), jnp.float32),
        ],
    )(x, W_router)


def _make_sc_topk_body(rows_per_w):
    def _sc_topk_body(scores_hbm, ew_hbm, ei_hbm, cnt_hbm, s_v, ew_v, ei_v,
                      hist_v, sem):
        c = lax.axis_index("c")
        s_id = lax.axis_index("s")
        wid = s_id * _NC + c
        base = wid * rows_per_w

        pltpu.sync_copy(scores_hbm.at[pl.ds(base, rows_per_w)], s_v)

        iota = lax.iota(jnp.int32, _L)
        lane_lt8 = iota < TOP_K
        zeros16 = jnp.zeros((_L,), jnp.int32)
        ones16 = jnp.ones((_L,), jnp.int32)
        for j in range(NUM_EXPERTS // _L):
            hist_v[pl.ds(_L * j, _L)] = zeros16

        def merge(ka, va, kb, vb, descending):
            mk = jnp.where(lane_lt8, ka, kb)
            mv = jnp.where(lane_lt8, va, vb)
            return plsc.sort_key_val(mk, mv, descending=descending)

        @plsc.parallel_loop(0, rows_per_w, 1, unroll=8)
        def row_body(r):
            ks, vs = [], []
            for j in range(NUM_EXPERTS // _L):
                kj = s_v[r, pl.ds(_L * j, _L)]
                sk, sv = plsc.sort_key_val(
                    kj, iota + _L * j, descending=(j % 2 == 0)
                )
                ks.append(sk)
                vs.append(sv)
            k01, v01 = merge(ks[0], vs[0], ks[1], vs[1], True)
            k23, v23 = merge(ks[2], vs[2], ks[3], vs[3], False)
            kf, vf = merge(k01, v01, k23, v23, True)
            out_idx = r * TOP_K + iota
            plsc.store_scatter(ew_v, [out_idx], kf, mask=lane_lt8)
            plsc.store_scatter(ei_v, [out_idx], vf, mask=lane_lt8)

        def hist_body(i, carry):
            r16 = i * (4 * _L)
            for u in range(4):
                v = ei_v[pl.ds(r16 + u * _L, _L)]
                plsc.addupdate_scatter(hist_v, [v], ones16)
            return carry

        lax.fori_loop(0, rows_per_w * TOP_K // (4 * _L), hist_body, 0)

        pltpu.sync_copy(ew_v, ew_hbm.at[pl.ds(base * TOP_K, rows_per_w * TOP_K)])
        pltpu.sync_copy(ei_v, ei_hbm.at[pl.ds(base * TOP_K, rows_per_w * TOP_K)])
        pltpu.sync_copy(hist_v, cnt_hbm.at[wid])

    return _sc_topk_body


@functools.cache
def _sc_topk(n_rows):
    rows_per_w = n_rows // _NW
    return pl.kernel(
        _make_sc_topk_body(rows_per_w),
        out_type=[
            jax.ShapeDtypeStruct((n_rows * TOP_K,), jnp.float32),
            jax.ShapeDtypeStruct((n_rows * TOP_K,), jnp.int32),
            jax.ShapeDtypeStruct((_NW, NUM_EXPERTS), jnp.int32),
        ],
        mesh=plsc.VectorSubcoreMesh(
            core_axis_name="c", subcore_axis_name="s",
            num_cores=_NC, num_subcores=_NS,
        ),
        compiler_params=pltpu.CompilerParams(needs_layout_passes=False),
        scratch_types=[
            pltpu.VMEM((rows_per_w, NUM_EXPERTS), jnp.float32),
            pltpu.VMEM((rows_per_w * TOP_K,), jnp.float32),
            pltpu.VMEM((rows_per_w * TOP_K,), jnp.int32),
            pltpu.VMEM((NUM_EXPERTS,), jnp.int32),
            pltpu.SemaphoreType.DMA,
        ],
    )


def kernel(x, W_router):
    chunk = N_TOKENS // N_CHUNKS
    louts, souts, ews, eis, cnts = [], [], [], [], []
    for i in range(N_CHUNKS):
        lg, sc = _tc_router(lax.slice(x, (i * chunk, 0), ((i + 1) * chunk, D_MODEL)), W_router)
        ew_flat, ei_flat, cnt_p = _sc_topk(chunk)(sc)
        louts.append(lg)
        souts.append(sc)
        ews.append(ew_flat.reshape(chunk, TOP_K))
        eis.append(ei_flat.reshape(chunk, TOP_K))
        cnts.append(cnt_p)
    if N_CHUNKS == 1:
        logits, scores, ew, ei = louts[0], souts[0], ews[0], eis[0]
    else:
        logits = jnp.concatenate(louts, axis=0)
        scores = jnp.concatenate(souts, axis=0)
        ew = jnp.concatenate(ews, axis=0)
        ei = jnp.concatenate(eis, axis=0)
    cnt = jnp.sum(jnp.stack(cnts), axis=(0, 1), dtype=jnp.int32)
    return (logits, scores, ew, ei, cnt)


# speedup vs baseline: 1.1362x; 1.0093x over previous
import functools

import jax
import jax.numpy as jnp
from jax import lax
from jax.experimental import pallas as pl
from jax.experimental.pallas import tpu as pltpu
from jax.experimental.pallas import tpu_sc as plsc

D_MODEL = 2048
NUM_EXPERTS = 64
TOP_K = 8
N_TOKENS = 16384

BLK = 1024

_NC = 2
_NS = 16
_NW = _NC * _NS
_L = 16


def _router_body(x_ref, w_ref, logits_ref, scores_ref):
    logits = jnp.dot(x_ref[...], w_ref[...], preferred_element_type=jnp.float32)
    logits_ref[...] = logits
    m = jnp.max(logits, axis=-1, keepdims=True)
    e = jnp.exp(logits - m)
    scores_ref[...] = e / jnp.sum(e, axis=-1, keepdims=True)


def _tc_router(x, W_router):
    n_rows = x.shape[0]
    n_blocks = n_rows // BLK
    return pl.pallas_call(
        _router_body,
        grid=(n_blocks,),
        in_specs=[
            pl.BlockSpec((BLK, D_MODEL), lambda i: (i, 0)),
            pl.BlockSpec((D_MODEL, NUM_EXPERTS), lambda i: (0, 0)),
        ],
        out_specs=[
            pl.BlockSpec((BLK, NUM_EXPERTS), lambda i: (i, 0)),
            pl.BlockSpec((BLK, NUM_EXPERTS), lambda i: (i, 0)),
        ],
        out_shape=[
            jax.ShapeDtypeStruct((n_rows, NUM_EXPERTS), jnp.float32),
            jax.ShapeDtypeStruct((n_rows, NUM_EXPERTS), jnp.float32),
        ],
    )(x, W_router)


def _make_sc_topk_body(rows_per_w):
    def _sc_topk_body(scores_hbm, ew_hbm, ei_hbm, cnt_hbm, s_v, ew_v, ei_v,
                      hist_v, sem):
        c = lax.axis_index("c")
        s_id = lax.axis_index("s")
        wid = s_id * _NC + c
        base = wid * rows_per_w

        pltpu.sync_copy(scores_hbm.at[pl.ds(base, rows_per_w)], s_v)

        iota = lax.iota(jnp.int32, _L)
        lane_lt8 = iota < TOP_K
        zeros16 = jnp.zeros((_L,), jnp.int32)
        ones16 = jnp.ones((_L,), jnp.int32)
        for j in range(NUM_EXPERTS // _L):
            hist_v[pl.ds(_L * j, _L)] = zeros16

        def merge(ka, va, kb, vb, descending):
            mk = jnp.where(lane_lt8, ka, kb)
            mv = jnp.where(lane_lt8, va, vb)
            return plsc.sort_key_val(mk, mv, descending=descending)

        @plsc.parallel_loop(0, rows_per_w, 1, unroll=8)
        def row_body(r):
            ks, vs = [], []
            for j in range(NUM_EXPERTS // _L):
                kj = s_v[r, pl.ds(_L * j, _L)]
                sk, sv = plsc.sort_key_val(
                    kj, iota + _L * j, descending=(j % 2 == 0)
                )
                ks.append(sk)
                vs.append(sv)
            k01, v01 = merge(ks[0], vs[0], ks[1], vs[1], True)
            k23, v23 = merge(ks[2], vs[2], ks[3], vs[3], False)
            kf, vf = merge(k01, v01, k23, v23, True)
            out_idx = r * TOP_K + iota
            plsc.store_scatter(ew_v, [out_idx], kf, mask=lane_lt8)
            plsc.store_scatter(ei_v, [out_idx], vf, mask=lane_lt8)

        def hist_body(i, carry):
            r16 = i * (4 * _L)
            for u in range(4):
                v = ei_v[pl.ds(r16 + u * _L, _L)]
                plsc.addupdate_scatter(hist_v, [v], ones16)
            return carry

        lax.fori_loop(0, rows_per_w * TOP_K // (4 * _L), hist_body, 0)

        pltpu.sync_copy(ew_v, ew_hbm.at[pl.ds(base * TOP_K, rows_per_w * TOP_K)])
        pltpu.sync_copy(ei_v, ei_hbm.at[pl.ds(base * TOP_K, rows_per_w * TOP_K)])
        pltpu.sync_copy(hist_v, cnt_hbm.at[wid])

    return _sc_topk_body


@functools.cache
def _sc_topk(n_rows):
    rows_per_w = n_rows // _NW
    return pl.kernel(
        _make_sc_topk_body(rows_per_w),
        out_type=[
            jax.ShapeDtypeStruct((n_rows * TOP_K,), jnp.float32),
            jax.ShapeDtypeStruct((n_rows * TOP_K,), jnp.int32),
            jax.ShapeDtypeStruct((_NW, NUM_EXPERTS), jnp.int32),
        ],
        mesh=plsc.VectorSubcoreMesh(
            core_axis_name="c", subcore_axis_name="s",
            num_cores=_NC, num_subcores=_NS,
        ),
        compiler_params=pltpu.CompilerParams(needs_layout_passes=False),
        scratch_types=[
            pltpu.VMEM((rows_per_w, NUM_EXPERTS), jnp.float32),
            pltpu.VMEM((rows_per_w * TOP_K,), jnp.float32),
            pltpu.VMEM((rows_per_w * TOP_K,), jnp.int32),
            pltpu.VMEM((NUM_EXPERTS,), jnp.int32),
            pltpu.SemaphoreType.DMA,
        ],
    )


def kernel(x, W_router):
    logits, scores = _tc_router(x, W_router)
    ew_flat, ei_flat, cnt_p = _sc_topk(N_TOKENS)(scores)
    ew = ew_flat.reshape(N_TOKENS, TOP_K)
    ei = ei_flat.reshape(N_TOKENS, TOP_K)
    cnt = jnp.sum(cnt_p, axis=0, dtype=jnp.int32)
    return (logits, scores, ew, ei, cnt)
